# Initial kernel scaffold; baseline (speedup 1.0000x reference)
import jax
import jax.numpy as jnp
from jax.experimental import pallas as pl


def kernel(node_type, embeddings):
    raise NotImplementedError("write your pallas kernel here")


# speedup vs baseline: 1.6786x; 1.6786x over previous
import jax
import jax.numpy as jnp
from jax import lax
from jax.experimental import pallas as pl
from jax.experimental.pallas import tpu as pltpu
from jax.experimental.pallas import tpu_sc as plsc

_D = 64
_B = 100000
_NC = 2
_NS = 16
_NW = _NC * _NS
_CH = 128
_FULL = _B // _CH
_TAIL = _B - _FULL * _CH
_BASE_N = _FULL // _NW
_EXTRA = _FULL - _BASE_N * _NW


def _gather_body(idx_hbm, tab_hbm, out_hbm, idx_v, rows_v, idx_t, rows_t, sem):
    c = lax.axis_index("c")
    s = lax.axis_index("s")
    wid = s * _NC + c
    n = _BASE_N + jnp.where(wid < _EXTRA, 1, 0)
    start = wid * _BASE_N + jnp.minimum(wid, _EXTRA)

    def body(t, carry):
        b = pl.multiple_of((start + t) * _CH, _CH)
        pltpu.sync_copy(idx_hbm.at[pl.ds(b, _CH)], idx_v)
        pltpu.async_copy(tab_hbm.at[idx_v], rows_v, sem).wait()
        pltpu.sync_copy(rows_v, out_hbm.at[pl.ds(b, _CH)])
        return carry

    lax.fori_loop(0, n, body, 0)

    @pl.when(wid == _NW - 1)
    def _tail():
        tb = _FULL * _CH
        pltpu.sync_copy(idx_hbm.at[pl.ds(tb, _TAIL)], idx_t)
        pltpu.async_copy(tab_hbm.at[idx_t], rows_t, sem).wait()
        pltpu.sync_copy(rows_t, out_hbm.at[pl.ds(tb, _TAIL)])


@jax.jit
def _embed_lookup(idx, table):
    f = pl.kernel(
        _gather_body,
        out_type=jax.ShapeDtypeStruct((_B, _D), jnp.float32),
        mesh=plsc.VectorSubcoreMesh(core_axis_name="c", subcore_axis_name="s"),
        scratch_types=[
            pltpu.VMEM((_CH,), jnp.int32),
            pltpu.VMEM((_CH, _D), jnp.float32),
            pltpu.VMEM((_TAIL,), jnp.int32),
            pltpu.VMEM((_TAIL, _D), jnp.float32),
            pltpu.SemaphoreType.DMA,
        ],
        compiler_params=pltpu.CompilerParams(use_tc_tiling_on_sc=False),
    )
    return f(idx, table)


def kernel(node_type, embeddings):
    idx = node_type.reshape(-1).astype(jnp.int32)
    return _embed_lookup(idx, embeddings.astype(jnp.float32))


# speedup vs baseline: 2.9037x; 1.7298x over previous
import jax
import jax.numpy as jnp
from jax import lax
from jax.experimental import pallas as pl
from jax.experimental.pallas import tpu as pltpu
from jax.experimental.pallas import tpu_sc as plsc

_D = 64
_B = 100000
_NC = 2
_NS = 16
_NW = _NC * _NS
_CH = 128
_NCHW = 25
_SPAN = _NCHW * _CH
_STRIDE_NUM = _B - _SPAN
_NB = 12
_LAG = 2


def _gather_body(idx_hbm, tab_hbm, out_hbm, tab_sh, idx_v, rows_v, gsem, wsem):
    c = lax.axis_index("c")
    s = lax.axis_index("s")
    wid = s * _NC + c

    @pl.when(s == 0)
    def _stage_table():
        pltpu.sync_copy(tab_hbm, tab_sh)

    base = jnp.bitwise_and((wid * _STRIDE_NUM) // (_NW - 1), -8)
    base = pl.multiple_of(base, 8)
    pltpu.sync_copy(idx_hbm.at[pl.ds(base, _SPAN)], idx_v)
    plsc.subcore_barrier()

    def _gather(t, slot):
        off = pl.multiple_of(t * _CH, _CH)
        return pltpu.make_async_copy(
            tab_sh.at[idx_v.at[pl.ds(off, _CH)]], rows_v.at[slot], gsem.at[slot])

    def _write(t, slot):
        off = pl.multiple_of(t * _CH, _CH)
        return pltpu.make_async_copy(
            rows_v.at[slot], out_hbm.at[pl.ds(base + off, _CH)], wsem.at[slot])

    def _step(t, carry):
        slot = lax.rem(t, _NB)

        @pl.when(t >= _NB)
        def _recycle():
            _write(t - _NB, slot).wait()

        _gather(t, slot).start()

        @pl.when(t >= _LAG)
        def _drain():
            u = t - _LAG
            uslot = lax.rem(u, _NB)
            _gather(u, uslot).wait()
            _write(u, uslot).start()

        return carry

    lax.fori_loop(0, _NCHW, _step, 0)

    def _tail_write(k, carry):
        u = _NCHW - _LAG + k
        uslot = lax.rem(u, _NB)
        _gather(u, uslot).wait()
        _write(u, uslot).start()
        return carry

    lax.fori_loop(0, _LAG, _tail_write, 0)

    def _tail_drain(k, carry):
        u = _NCHW - _NB + k
        uslot = lax.rem(u, _NB)
        _write(u, uslot).wait()
        return carry

    lax.fori_loop(0, _NB, _tail_drain, 0)


@jax.jit
def _embed_lookup(idx, table):
    f = pl.kernel(
        _gather_body,
        out_type=jax.ShapeDtypeStruct((_B, _D), jnp.float32),
        mesh=plsc.VectorSubcoreMesh(core_axis_name="c", subcore_axis_name="s"),
        scratch_types=[
            pltpu.VMEM_SHARED((100, _D), jnp.float32),
            pltpu.VMEM((_SPAN,), jnp.int32),
            pltpu.VMEM((_NB, _CH, _D), jnp.float32),
            pltpu.SemaphoreType.DMA((_NB,)),
            pltpu.SemaphoreType.DMA((_NB,)),
        ],
        compiler_params=pltpu.CompilerParams(use_tc_tiling_on_sc=False),
    )
    return f(idx, table)


def kernel(node_type, embeddings):
    idx = node_type.reshape(-1).astype(jnp.int32)
    return _embed_lookup(idx, embeddings.astype(jnp.float32))
